# baseline (speedup 1.0000x reference)
import functools

import jax
import jax.numpy as jnp
from jax import lax
from jax.experimental import pallas as pl
from jax.experimental.pallas import tpu as pltpu
from jax.experimental.pallas import tpu_sc as plsc

NUM_CORES = 2
NUM_SUBCORES = 16
NUM_WORKERS = NUM_CORES * NUM_SUBCORES

CHUNK = 512


def _make_lookup(total, dim):
  assert total % (NUM_WORKERS * CHUNK) == 0
  b_per_w = total // NUM_WORKERS
  n_chunks = b_per_w // CHUNK
  mesh = plsc.VectorSubcoreMesh(core_axis_name="c", subcore_axis_name="s")

  @functools.partial(
      pl.kernel,
      out_type=jax.ShapeDtypeStruct((total, dim), jnp.float32),
      mesh=mesh,
      scratch_types=[
          pltpu.VMEM((2, CHUNK), jnp.int32),
          pltpu.VMEM((2, CHUNK, dim), jnp.float32),
          pltpu.SemaphoreType.DMA((2,)),
          pltpu.SemaphoreType.DMA((2,)),
      ],
      compiler_params=pltpu.CompilerParams(use_tc_tiling_on_sc=False),
  )
  def lookup(idx_hbm, table_hbm, out_hbm, idx_v, rows_v, idx_sem, row_sem):
    wid = lax.axis_index("s") * NUM_CORES + lax.axis_index("c")
    base = wid * b_per_w

    def idx_load(slot, chunk):
      return pltpu.make_async_copy(
          idx_hbm.at[pl.ds(base + chunk * CHUNK, CHUNK)],
          idx_v.at[slot],
          idx_sem.at[slot],
      )

    def gather(slot):
      return pltpu.make_async_copy(
          table_hbm.at[idx_v.at[slot]],
          rows_v.at[slot],
          row_sem.at[slot],
      )

    idx_load(0, 0).start()

    def body(c, _):
      slot = lax.rem(c, 2)
      nxt = 1 - slot
      @pl.when(c + 1 < n_chunks)
      def _():
        idx_load(nxt, c + 1).start()

      idx_load(slot, c).wait()
      gather(slot).start()
      gather(slot).wait()
      pltpu.sync_copy(
          rows_v.at[slot],
          out_hbm.at[pl.ds(base + c * CHUNK, CHUNK)],
      )
      return ()

    lax.fori_loop(0, n_chunks, body, (), unroll=False)

  return lookup


def kernel(indices, table):
  b, s = indices.shape
  total = b * s
  dim = table.shape[1]
  flat_idx = indices.reshape(total).astype(jnp.int32)
  out = _make_lookup(total, dim)(flat_idx, table)
  return out.reshape(b, s, dim)


# speedup vs baseline: 1.0203x; 1.0203x over previous
import functools

import jax
import jax.numpy as jnp
from jax import lax
from jax.experimental import pallas as pl
from jax.experimental.pallas import tpu as pltpu
from jax.experimental.pallas import tpu_sc as plsc

NUM_CORES = 2
NUM_SUBCORES = 16
NUM_WORKERS = NUM_CORES * NUM_SUBCORES

CHUNK = 512
NBUF = 3


def _make_lookup(total, dim):
  assert total % (NUM_WORKERS * CHUNK) == 0
  b_per_w = total // NUM_WORKERS
  n_chunks = b_per_w // CHUNK
  assert n_chunks >= NBUF
  mesh = plsc.VectorSubcoreMesh(core_axis_name="c", subcore_axis_name="s")

  @functools.partial(
      pl.kernel,
      out_type=jax.ShapeDtypeStruct((total, dim), jnp.float32),
      mesh=mesh,
      scratch_types=[
          pltpu.VMEM((NBUF, CHUNK), jnp.int32),
          pltpu.VMEM((NBUF, CHUNK, dim), jnp.float32),
          pltpu.SemaphoreType.DMA((NBUF,)),
          pltpu.SemaphoreType.DMA((NBUF,)),
          pltpu.SemaphoreType.DMA((NBUF,)),
      ],
      compiler_params=pltpu.CompilerParams(use_tc_tiling_on_sc=False),
  )
  def lookup(idx_hbm, table_hbm, out_hbm, idx_v, rows_v, idx_sem, row_sem,
             out_sem):
    wid = lax.axis_index("s") * NUM_CORES + lax.axis_index("c")
    base = wid * b_per_w

    def idx_load(slot, chunk):
      return pltpu.make_async_copy(
          idx_hbm.at[pl.ds(base + chunk * CHUNK, CHUNK)],
          idx_v.at[slot],
          idx_sem.at[slot],
      )

    def gather(slot):
      return pltpu.make_async_copy(
          table_hbm.at[idx_v.at[slot]],
          rows_v.at[slot],
          row_sem.at[slot],
      )

    def store(slot, chunk):
      return pltpu.make_async_copy(
          rows_v.at[slot],
          out_hbm.at[pl.ds(base + chunk * CHUNK, CHUNK)],
          out_sem.at[slot],
      )

    for b in range(NBUF):
      idx_load(b, b).start()

    def body(c, _):
      slot = lax.rem(c, NBUF)
      idx_load(slot, c).wait()
      @pl.when(c >= NBUF)
      def _():
        store(slot, c - NBUF).wait()

      gather(slot).start()
      gather(slot).wait()
      store(slot, c).start()
      @pl.when(c + NBUF < n_chunks)
      def _():
        idx_load(slot, c + NBUF).start()

      return ()

    lax.fori_loop(0, n_chunks, body, (), unroll=False)

    for k in range(NBUF):
      c = n_chunks - NBUF + k
      store(c % NBUF, c).wait()

  return lookup


def kernel(indices, table):
  b, s = indices.shape
  total = b * s
  dim = table.shape[1]
  flat_idx = indices.reshape(total).astype(jnp.int32)
  out = _make_lookup(total, dim)(flat_idx, table)
  return out.reshape(b, s, dim)
